# baseline (speedup 1.0000x reference)
import functools

import jax
import jax.numpy as jnp
from jax import lax
from jax.experimental import pallas as pl
from jax.experimental.pallas import tpu as pltpu
from jax.experimental.pallas import tpu_sc as plsc

NUM_WORDS = 1000000
DIM_EMBED = 32
NUM_CLASSES = 2
BATCH = 4096
SEQ = 200

NW = 32
CHUNK = 100
CHUNKS_PER_ROW = SEQ // CHUNK
ROWS_PER_W = BATCH // NW
CHUNKS_PER_W = ROWS_PER_W * CHUNKS_PER_ROW
HALF = 16
NBUF = 4
STREAM = 112
PROJ_BLK = 1024


def _tc_project(table_t, w):
    grid = (NUM_WORDS + PROJ_BLK - 1) // PROJ_BLK

    def body(w_ref, t_ref, o0_ref, o1_ref):
        p = jnp.dot(w_ref[:], t_ref[:], preferred_element_type=jnp.float32)
        o0_ref[:] = p[0]
        o1_ref[:] = p[1]

    return pl.pallas_call(
        body,
        grid=(grid,),
        in_specs=[
            pl.BlockSpec((NUM_CLASSES, DIM_EMBED), lambda j: (0, 0)),
            pl.BlockSpec((DIM_EMBED, PROJ_BLK), lambda j: (0, j)),
        ],
        out_specs=[
            pl.BlockSpec((PROJ_BLK,), lambda j: (j,)),
            pl.BlockSpec((PROJ_BLK,), lambda j: (j,)),
        ],
        out_shape=[
            jax.ShapeDtypeStruct((NUM_WORDS,), jnp.float32),
            jax.ShapeDtypeStruct((NUM_WORDS,), jnp.float32),
        ],
    )(w, table_t)


def _sc_gather_sums(x_pad, p0, p1):
    mesh = plsc.VectorSubcoreMesh(core_axis_name="c", subcore_axis_name="s")

    @functools.partial(
        pl.kernel,
        out_type=jax.ShapeDtypeStruct((NW * CHUNKS_PER_W * 2 * HALF,),
                                      jnp.float32),
        mesh=mesh,
        scratch_types=[
            pltpu.VMEM((CHUNKS_PER_W * 128,), jnp.int32),
            pltpu.VMEM((NBUF, 2, STREAM), jnp.float32),
            pltpu.VMEM((CHUNKS_PER_W * 2 * HALF,), jnp.float32),
            pltpu.SemaphoreType.DMA((NBUF,)),
        ],
        compiler_params=pltpu.CompilerParams(use_tc_tiling_on_sc=False),
    )
    def k(x_hbm, p0_hbm, p1_hbm, out_hbm, idx_v, vals_v, sums_v, sem):
        wid = lax.axis_index("s") * 2 + lax.axis_index("c")
        base = wid * CHUNKS_PER_W * 128
        pltpu.sync_copy(x_hbm.at[pl.ds(base, CHUNKS_PER_W * 128)], idx_v)

        tail_mask = lax.iota(jnp.int32, HALF) < (CHUNK % HALF)

        def gather(t, b):
            isl = idx_v.at[pl.ds(t * 128, STREAM)]
            pltpu.make_async_copy(
                p0_hbm.at[isl], vals_v.at[b, 0], sem.at[b]).start()
            pltpu.make_async_copy(
                p1_hbm.at[isl], vals_v.at[b, 1], sem.at[b]).start()

        def drain(t, b):
            isl = idx_v.at[pl.ds(t * 128, STREAM)]
            pltpu.make_async_copy(
                p0_hbm.at[isl], vals_v.at[b, 0], sem.at[b]).wait()
            pltpu.make_async_copy(
                p1_hbm.at[isl], vals_v.at[b, 1], sem.at[b]).wait()

        for b in range(NBUF):
            gather(b, b)

        def group_body(g, _):
            t0 = g * NBUF
            for b in range(NBUF):
                t = t0 + b
                drain(t, b)
                acc = [jnp.zeros((HALF,), jnp.float32) for _ in range(4)]
                for c in range(2):
                    for u in range(STREAM // HALF):
                        v = vals_v[b, c, pl.ds(u * HALF, HALF)]
                        if (u + 1) * HALF > CHUNK:
                            v = jnp.where(tail_mask, v, 0.0)
                        acc[u % 2 + 2 * c] = acc[u % 2 + 2 * c] + v

                @pl.when(g < CHUNKS_PER_W // NBUF - 1)
                def _():
                    gather(t + NBUF, b)

                sums_v[pl.ds(t * 2 * HALF, HALF)] = acc[0] + acc[1]
                sums_v[pl.ds(t * 2 * HALF + HALF, HALF)] = acc[2] + acc[3]
            return 0

        lax.fori_loop(0, CHUNKS_PER_W // NBUF, group_body, 0)
        pltpu.sync_copy(
            sums_v,
            out_hbm.at[pl.ds(wid * CHUNKS_PER_W * 2 * HALF,
                             CHUNKS_PER_W * 2 * HALF)])

    return k(x_pad, p0, p1)


def _tc_fold(sums, bias):
    def body(s_ref, b_ref, o_ref):
        s = s_ref[:]
        n = BATCH * CHUNKS_PER_ROW
        c0 = jnp.sum(s[:, :HALF], axis=1)
        c1 = jnp.sum(s[:, HALF:], axis=1)
        c = jnp.stack([c0, c1], axis=1)
        c = jnp.reshape(c, (BATCH, CHUNKS_PER_ROW, NUM_CLASSES))
        o_ref[:] = jnp.sum(c, axis=1) * (1.0 / SEQ) + b_ref[:]

    return pl.pallas_call(
        body,
        out_shape=jax.ShapeDtypeStruct((BATCH, NUM_CLASSES), jnp.float32),
    )(sums, bias)


def kernel(x, embedding_table, fc_weight, fc_bias):
    p0, p1 = _tc_project(embedding_table.T, fc_weight)
    x_flat = jnp.reshape(x.astype(jnp.int32), (-1, CHUNK))
    x_pad = jnp.reshape(jnp.pad(x_flat, ((0, 0), (0, 128 - CHUNK))), (-1,))
    sums = _sc_gather_sums(x_pad, p0, p1)
    sums2 = jnp.reshape(sums, (BATCH * CHUNKS_PER_ROW, 2 * HALF))
    return _tc_fold(sums2, jnp.reshape(fc_bias, (1, NUM_CLASSES)))
